# baseline (speedup 1.0000x reference)
import functools

import jax
import jax.numpy as jnp
from jax import lax
from jax.experimental import pallas as pl
from jax.experimental.pallas import tpu as pltpu
from jax.experimental.pallas import tpu_sc as plsc

NB = 256
L = 16
NC, NS = 2, 16
NW = NC * NS
H, W = 2048, 2048
N = H * W
PER_W = N // NW
CHUNK = 32768
NCHUNK = PER_W // CHUNK
UNROLL = 8
UNROLL_HIST = 8
UNROLL_REMAP = 8

_mesh = plsc.VectorSubcoreMesh(
    core_axis_name="c", subcore_axis_name="s", num_cores=NC, num_subcores=NS
)


def _wid():
    return lax.axis_index("s") * NC + lax.axis_index("c")


def _reduce_minmax(mm_vmem, tmp):
    mnv = mm_vmem[pl.ds(0, L)]
    mxv = mm_vmem[pl.ds(L, L)]
    for i in range(1, NW):
        mnv = jnp.minimum(mnv, mm_vmem[pl.ds(2 * i * L, L)])
        mxv = jnp.maximum(mxv, mm_vmem[pl.ds((2 * i + 1) * L, L)])
    iota = lax.iota(jnp.int32, L)
    for s in (8, 4, 2, 1):
        tmp[pl.ds(0, L)] = mnv
        tmp[pl.ds(L, L)] = mxv
        idx = iota ^ s
        mnv = jnp.minimum(mnv, plsc.load_gather(tmp, [idx]))
        mxv = jnp.maximum(mxv, plsc.load_gather(tmp, [idx + L]))
    return mnv, mxv


@functools.partial(
    pl.kernel,
    out_type=jax.ShapeDtypeStruct((2 * NW * L,), jnp.float32),
    mesh=_mesh,
    compiler_params=pltpu.CompilerParams(needs_layout_passes=False),
    scratch_types=[
        pltpu.VMEM((CHUNK,), jnp.float32),
        pltpu.VMEM((CHUNK,), jnp.float32),
        pltpu.VMEM((2 * L,), jnp.float32),
        pltpu.SemaphoreType.DMA,
        pltpu.SemaphoreType.DMA,
    ],
)
def _minmax(x_hbm, out_hbm, buf0, buf1, mmbuf, sem0, sem1):
    wid = _wid()
    base = wid * PER_W
    bufs, sems = (buf0, buf1), (sem0, sem1)

    descs = [
        pltpu.async_copy(x_hbm.at[pl.ds(base + ci * CHUNK, CHUNK)],
                         bufs[ci], sems[ci])
        for ci in range(min(2, NCHUNK))
    ]

    mnv = jnp.full((L,), jnp.inf, jnp.float32)
    mxv = jnp.full((L,), -jnp.inf, jnp.float32)
    for ci in range(NCHUNK):
        cur = ci % 2
        descs[cur].wait()
        buf = bufs[cur]

        def _mm_body(j, c, buf=buf):
            mn, mx = c
            v = buf[pl.ds(j * L, L)]
            return jnp.minimum(mn, v), jnp.maximum(mx, v)

        mnv, mxv = plsc.parallel_loop(
            0, CHUNK // L, 1, unroll=UNROLL, carry=(mnv, mxv))(_mm_body)
        if ci + 2 < NCHUNK:
            descs[cur] = pltpu.async_copy(
                x_hbm.at[pl.ds(base + (ci + 2) * CHUNK, CHUNK)], buf, sems[cur])

    mmbuf[pl.ds(0, L)] = mnv
    mmbuf[pl.ds(L, L)] = mxv
    pltpu.sync_copy(mmbuf, out_hbm.at[pl.ds(wid * 2 * L, 2 * L)])


@functools.partial(
    pl.kernel,
    out_type=jax.ShapeDtypeStruct((NW * NB,), jnp.float32),
    mesh=_mesh,
    compiler_params=pltpu.CompilerParams(needs_layout_passes=False),
    scratch_types=[
        pltpu.VMEM((CHUNK,), jnp.float32),
        pltpu.VMEM((CHUNK,), jnp.float32),
        pltpu.VMEM((2 * NW * L,), jnp.float32),
        pltpu.VMEM((L * NB,), jnp.float32),
        pltpu.VMEM((L * NB,), jnp.float32),
        pltpu.VMEM((NB,), jnp.float32),
        pltpu.SemaphoreType.DMA,
        pltpu.SemaphoreType.DMA,
    ],
)
def _hist(x_hbm, mm_hbm, out_hbm, buf0, buf1, mmv, priv, priv2, histbuf,
          sem0, sem1):
    wid = _wid()
    base = wid * PER_W
    bufs, sems = (buf0, buf1), (sem0, sem1)

    descs = [
        pltpu.async_copy(x_hbm.at[pl.ds(base + ci * CHUNK, CHUNK)],
                         bufs[ci], sems[ci])
        for ci in range(min(2, NCHUNK))
    ]

    pltpu.sync_copy(mm_hbm, mmv)
    mn, mx = _reduce_minmax(mmv, histbuf)
    scale = NB / jnp.maximum(mx - mn, 1e-30)

    zeros = jnp.zeros((L,), jnp.float32)

    def zero_body(k, _):
        for u in range(UNROLL):
            priv[pl.ds((k * UNROLL + u) * L, L)] = zeros
            priv2[pl.ds((k * UNROLL + u) * L, L)] = zeros
        return 0

    lax.fori_loop(0, (L * NB) // (L * UNROLL), zero_body, 0)

    lane = lax.iota(jnp.int32, L)
    laneoff = lane * NB
    ones = jnp.ones((L,), jnp.float32)

    for ci in range(NCHUNK):
        cur = ci % 2
        descs[cur].wait()
        buf = bufs[cur]

        def _hist_body(j, buf=buf):
            v = buf[pl.ds(2 * j * L, L)]
            t = (v - mn) * scale
            ti = jnp.minimum(t.astype(jnp.int32), NB - 1)
            plsc.addupdate_scatter(priv, [laneoff + ti], ones)
            v2 = buf[pl.ds((2 * j + 1) * L, L)]
            t2 = (v2 - mn) * scale
            ti2 = jnp.minimum(t2.astype(jnp.int32), NB - 1)
            plsc.addupdate_scatter(priv2, [laneoff + ti2], ones)

        plsc.parallel_loop(0, CHUNK // (2 * L), 1,
                           unroll=UNROLL_HIST // 2)(_hist_body)
        if ci + 2 < NCHUNK:
            descs[cur] = pltpu.async_copy(
                x_hbm.at[pl.ds(base + (ci + 2) * CHUNK, CHUNK)], buf, sems[cur])

    for j in range(NB // L):
        acc = priv[pl.ds(j * L, L)] + priv2[pl.ds(j * L, L)]
        for l in range(1, L):
            acc = acc + priv[pl.ds(l * NB + j * L, L)]
            acc = acc + priv2[pl.ds(l * NB + j * L, L)]
        histbuf[pl.ds(j * L, L)] = acc
    pltpu.sync_copy(histbuf, out_hbm.at[pl.ds(wid * NB, NB)])


@functools.partial(
    pl.kernel,
    out_type=jax.ShapeDtypeStruct((N,), jnp.float32),
    mesh=_mesh,
    compiler_params=pltpu.CompilerParams(needs_layout_passes=False),
    scratch_types=[
        pltpu.VMEM((CHUNK,), jnp.float32),
        pltpu.VMEM((CHUNK,), jnp.float32),
        pltpu.VMEM((2 * NW * L,), jnp.float32),
        pltpu.VMEM((NW * NB,), jnp.float32),
        pltpu.VMEM((NB,), jnp.float32),
        pltpu.VMEM((NB,), jnp.float32),
        pltpu.SemaphoreType.DMA,
        pltpu.SemaphoreType.DMA,
        pltpu.SemaphoreType.DMA,
        pltpu.SemaphoreType.DMA,
    ],
)
def _remap(x_hbm, hp_hbm, mm_hbm, out_hbm, bin0, bin1,
           mmv, hpart, ctab, ctab2, isem0, isem1, osem0, osem1):
    wid = _wid()
    base = wid * PER_W
    ibufs, isems = (bin0, bin1), (isem0, isem1)
    osems = (osem0, osem1)

    idescs = [
        pltpu.async_copy(x_hbm.at[pl.ds(base + ci * CHUNK, CHUNK)],
                         ibufs[ci], isems[ci])
        for ci in range(min(2, NCHUNK))
    ]
    odescs = [None, None]

    pltpu.sync_copy(mm_hbm, mmv)
    pltpu.sync_copy(hp_hbm, hpart)
    mn, mx = _reduce_minmax(mmv, ctab)
    scale = NB / jnp.maximum(mx - mn, 1e-30)

    for j in range(NB // L):
        acc = hpart[pl.ds(j * L, L)]
        for w in range(1, NW):
            acc = acc + hpart[pl.ds(w * NB + j * L, L)]
        ctab[pl.ds(j * L, L)] = acc

    iota = lax.iota(jnp.int32, L)
    a_ref, b_ref = ctab, ctab2
    for s in (1, 2, 4, 8, 16, 32, 64, 128):
        for j in range(NB // L):
            cur = a_ref[pl.ds(j * L, L)]
            if j * L >= s:
                sh = a_ref[pl.ds(j * L - s, L)]
            elif (j + 1) * L <= s:
                sh = jnp.zeros((L,), jnp.float32)
            else:
                idx = jnp.maximum(iota - s, 0)
                val = plsc.load_gather(a_ref, [idx])
                sh = jnp.where(iota >= s, val, 0.0)
            b_ref[pl.ds(j * L, L)] = cur + sh
        a_ref, b_ref = b_ref, a_ref

    inv = jnp.float32(1.0 / N)
    for j in range(NB // L):
        a_ref[pl.ds(j * L, L)] = a_ref[pl.ds(j * L, L)] * inv

    iota_next = iota + 1
    for j in range(NB // L):
        nxt = plsc.load_gather(
            a_ref, [jnp.minimum(j * L + iota_next, NB - 1)])
        b_ref[pl.ds(j * L, L)] = nxt - a_ref[pl.ds(j * L, L)]

    intercept = jnp.float32(-0.5) - mn * scale

    for ci in range(NCHUNK):
        cur = ci % 2
        off = base + ci * CHUNK
        idescs[cur].wait()
        buf = ibufs[cur]

        def _remap_body(j, buf=buf):
            v = buf[pl.ds(j * L, L)]
            t = jnp.clip(v * scale + intercept, 0.0, float(NB - 1))
            ti = jnp.minimum(t.astype(jnp.int32), NB - 2)
            f = t - ti.astype(jnp.float32)
            c0 = plsc.load_gather(a_ref, [ti])
            d = plsc.load_gather(b_ref, [ti])
            buf[pl.ds(j * L, L)] = c0 + f * d

        plsc.parallel_loop(0, CHUNK // L, 1, unroll=UNROLL_REMAP)(_remap_body)
        odescs[cur] = pltpu.async_copy(
            buf, out_hbm.at[pl.ds(off, CHUNK)], osems[cur])
        if ci + 2 < NCHUNK:
            odescs[cur].wait()
            odescs[cur] = None
            idescs[cur] = pltpu.async_copy(
                x_hbm.at[pl.ds(base + (ci + 2) * CHUNK, CHUNK)], buf, isems[cur])

    for d in odescs:
        if d is not None:
            d.wait()


def kernel(x):
    xt = x.reshape(H // 8, 8, W // 128, 128).transpose(0, 2, 1, 3).reshape(-1)
    mm = _minmax(xt)
    hp = _hist(xt, mm)
    y = _remap(xt, hp, mm)
    return (y.reshape(H // 8, W // 128, 8, 128)
             .transpose(0, 2, 1, 3).reshape(1, H, W))


# speedup vs baseline: 1.0422x; 1.0422x over previous
import functools

import jax
import jax.numpy as jnp
from jax import lax
from jax.experimental import pallas as pl
from jax.experimental.pallas import tpu as pltpu
from jax.experimental.pallas import tpu_sc as plsc

NB = 256
L = 16
NC, NS = 2, 16
NW = NC * NS
H, W = 2048, 2048
N = H * W
PER_W = N // NW
CHUNK = 32768
CHUNK_R = 16384
NCHUNK = PER_W // CHUNK
NCHUNK_R = PER_W // CHUNK_R
UNROLL = 8
UNROLL_HIST = 8
UNROLL_REMAP = 8

_mesh = plsc.VectorSubcoreMesh(
    core_axis_name="c", subcore_axis_name="s", num_cores=NC, num_subcores=NS
)


def _wid():
    return lax.axis_index("s") * NC + lax.axis_index("c")


def _reduce_minmax(mm_vmem, tmp):
    mnv = mm_vmem[pl.ds(0, L)]
    mxv = mm_vmem[pl.ds(L, L)]
    for i in range(1, NW):
        mnv = jnp.minimum(mnv, mm_vmem[pl.ds(2 * i * L, L)])
        mxv = jnp.maximum(mxv, mm_vmem[pl.ds((2 * i + 1) * L, L)])
    iota = lax.iota(jnp.int32, L)
    for s in (8, 4, 2, 1):
        tmp[pl.ds(0, L)] = mnv
        tmp[pl.ds(L, L)] = mxv
        idx = iota ^ s
        mnv = jnp.minimum(mnv, plsc.load_gather(tmp, [idx]))
        mxv = jnp.maximum(mxv, plsc.load_gather(tmp, [idx + L]))
    return mnv, mxv


@functools.partial(
    pl.kernel,
    out_type=jax.ShapeDtypeStruct((2 * NW * L,), jnp.float32),
    mesh=_mesh,
    compiler_params=pltpu.CompilerParams(needs_layout_passes=False),
    scratch_types=[
        pltpu.VMEM((CHUNK,), jnp.float32),
        pltpu.VMEM((CHUNK,), jnp.float32),
        pltpu.VMEM((2 * L,), jnp.float32),
        pltpu.SemaphoreType.DMA,
        pltpu.SemaphoreType.DMA,
    ],
)
def _minmax(x_hbm, out_hbm, buf0, buf1, mmbuf, sem0, sem1):
    wid = _wid()
    base = wid * PER_W
    bufs, sems = (buf0, buf1), (sem0, sem1)

    descs = [
        pltpu.async_copy(x_hbm.at[pl.ds(base + ci * CHUNK, CHUNK)],
                         bufs[ci], sems[ci])
        for ci in range(min(2, NCHUNK))
    ]

    mnv = jnp.full((L,), jnp.inf, jnp.float32)
    mxv = jnp.full((L,), -jnp.inf, jnp.float32)
    for ci in range(NCHUNK):
        cur = ci % 2
        descs[cur].wait()
        buf = bufs[cur]

        def _mm_body(j, c, buf=buf):
            mn, mx = c
            v = buf[pl.ds(j * L, L)]
            return jnp.minimum(mn, v), jnp.maximum(mx, v)

        mnv, mxv = plsc.parallel_loop(
            0, CHUNK // L, 1, unroll=UNROLL, carry=(mnv, mxv))(_mm_body)
        if ci + 2 < NCHUNK:
            descs[cur] = pltpu.async_copy(
                x_hbm.at[pl.ds(base + (ci + 2) * CHUNK, CHUNK)], buf, sems[cur])

    mmbuf[pl.ds(0, L)] = mnv
    mmbuf[pl.ds(L, L)] = mxv
    pltpu.sync_copy(mmbuf, out_hbm.at[pl.ds(wid * 2 * L, 2 * L)])


@functools.partial(
    pl.kernel,
    out_type=jax.ShapeDtypeStruct((NW * NB,), jnp.float32),
    mesh=_mesh,
    compiler_params=pltpu.CompilerParams(needs_layout_passes=False),
    scratch_types=[
        pltpu.VMEM((CHUNK,), jnp.float32),
        pltpu.VMEM((CHUNK,), jnp.float32),
        pltpu.VMEM((2 * NW * L,), jnp.float32),
        pltpu.VMEM((L * NB,), jnp.float32),
        pltpu.VMEM((L * NB,), jnp.float32),
        pltpu.VMEM((NB,), jnp.float32),
        pltpu.SemaphoreType.DMA,
        pltpu.SemaphoreType.DMA,
    ],
)
def _hist(x_hbm, mm_hbm, out_hbm, buf0, buf1, mmv, priv, priv2, histbuf,
          sem0, sem1):
    wid = _wid()
    base = wid * PER_W
    bufs, sems = (buf0, buf1), (sem0, sem1)

    descs = [
        pltpu.async_copy(x_hbm.at[pl.ds(base + ci * CHUNK, CHUNK)],
                         bufs[ci], sems[ci])
        for ci in range(min(2, NCHUNK))
    ]

    pltpu.sync_copy(mm_hbm, mmv)
    mn, mx = _reduce_minmax(mmv, histbuf)
    scale = NB / jnp.maximum(mx - mn, 1e-30)

    zeros = jnp.zeros((L,), jnp.float32)

    def zero_body(k, _):
        for u in range(UNROLL):
            priv[pl.ds((k * UNROLL + u) * L, L)] = zeros
            priv2[pl.ds((k * UNROLL + u) * L, L)] = zeros
        return 0

    lax.fori_loop(0, (L * NB) // (L * UNROLL), zero_body, 0)

    lane = lax.iota(jnp.int32, L)
    laneoff = lane * NB
    ones = jnp.ones((L,), jnp.float32)

    for ci in range(NCHUNK):
        cur = ci % 2
        descs[cur].wait()
        buf = bufs[cur]

        def _hist_body(j, buf=buf):
            v = buf[pl.ds(2 * j * L, L)]
            t = (v - mn) * scale
            ti = jnp.minimum(t.astype(jnp.int32), NB - 1)
            plsc.addupdate_scatter(priv, [laneoff + ti], ones)
            v2 = buf[pl.ds((2 * j + 1) * L, L)]
            t2 = (v2 - mn) * scale
            ti2 = jnp.minimum(t2.astype(jnp.int32), NB - 1)
            plsc.addupdate_scatter(priv2, [laneoff + ti2], ones)

        plsc.parallel_loop(0, CHUNK // (2 * L), 1,
                           unroll=UNROLL_HIST // 2)(_hist_body)
        if ci + 2 < NCHUNK:
            descs[cur] = pltpu.async_copy(
                x_hbm.at[pl.ds(base + (ci + 2) * CHUNK, CHUNK)], buf, sems[cur])

    for j in range(NB // L):
        acc = priv[pl.ds(j * L, L)] + priv2[pl.ds(j * L, L)]
        for l in range(1, L):
            acc = acc + priv[pl.ds(l * NB + j * L, L)]
            acc = acc + priv2[pl.ds(l * NB + j * L, L)]
        histbuf[pl.ds(j * L, L)] = acc
    pltpu.sync_copy(histbuf, out_hbm.at[pl.ds(wid * NB, NB)])


@functools.partial(
    pl.kernel,
    out_type=jax.ShapeDtypeStruct((N,), jnp.float32),
    mesh=_mesh,
    compiler_params=pltpu.CompilerParams(needs_layout_passes=False),
    scratch_types=[
        pltpu.VMEM((CHUNK_R,), jnp.float32),
        pltpu.VMEM((CHUNK_R,), jnp.float32),
        pltpu.VMEM((CHUNK_R,), jnp.float32),
        pltpu.VMEM((CHUNK_R,), jnp.float32),
        pltpu.VMEM((2 * NW * L,), jnp.float32),
        pltpu.VMEM((NW * NB,), jnp.float32),
        pltpu.VMEM((NB,), jnp.float32),
        pltpu.VMEM((NB,), jnp.float32),
        pltpu.SemaphoreType.DMA,
        pltpu.SemaphoreType.DMA,
        pltpu.SemaphoreType.DMA,
        pltpu.SemaphoreType.DMA,
    ],
)
def _remap(x_hbm, hp_hbm, mm_hbm, out_hbm, bin0, bin1, bout0, bout1,
           mmv, hpart, ctab, ctab2, isem0, isem1, osem0, osem1):
    wid = _wid()
    base = wid * PER_W
    ibufs, isems = (bin0, bin1), (isem0, isem1)
    obufs, osems = (bout0, bout1), (osem0, osem1)

    idescs = [
        pltpu.async_copy(x_hbm.at[pl.ds(base + ci * CHUNK_R, CHUNK_R)],
                         ibufs[ci], isems[ci])
        for ci in range(min(2, NCHUNK_R))
    ]
    odescs = [None, None]

    pltpu.sync_copy(mm_hbm, mmv)
    pltpu.sync_copy(hp_hbm, hpart)
    mn, mx = _reduce_minmax(mmv, ctab)
    scale = NB / jnp.maximum(mx - mn, 1e-30)

    for j in range(NB // L):
        acc = hpart[pl.ds(j * L, L)]
        for w in range(1, NW):
            acc = acc + hpart[pl.ds(w * NB + j * L, L)]
        ctab[pl.ds(j * L, L)] = acc

    iota = lax.iota(jnp.int32, L)
    a_ref, b_ref = ctab, ctab2
    for s in (1, 2, 4, 8, 16, 32, 64, 128):
        for j in range(NB // L):
            cur = a_ref[pl.ds(j * L, L)]
            if j * L >= s:
                sh = a_ref[pl.ds(j * L - s, L)]
            elif (j + 1) * L <= s:
                sh = jnp.zeros((L,), jnp.float32)
            else:
                idx = jnp.maximum(iota - s, 0)
                val = plsc.load_gather(a_ref, [idx])
                sh = jnp.where(iota >= s, val, 0.0)
            b_ref[pl.ds(j * L, L)] = cur + sh
        a_ref, b_ref = b_ref, a_ref

    inv = jnp.float32(1.0 / N)
    for j in range(NB // L):
        a_ref[pl.ds(j * L, L)] = a_ref[pl.ds(j * L, L)] * inv

    iota_next = iota + 1
    for j in range(NB // L):
        nxt = plsc.load_gather(
            a_ref, [jnp.minimum(j * L + iota_next, NB - 1)])
        b_ref[pl.ds(j * L, L)] = nxt - a_ref[pl.ds(j * L, L)]

    intercept = jnp.float32(-0.5) - mn * scale

    for ci in range(NCHUNK_R):
        cur = ci % 2
        off = base + ci * CHUNK_R
        idescs[cur].wait()
        if odescs[cur] is not None:
            odescs[cur].wait()
        bufin, bufout = ibufs[cur], obufs[cur]

        def _remap_body(j, bufin=bufin, bufout=bufout):
            v = bufin[pl.ds(j * L, L)]
            t = jnp.clip(v * scale + intercept, 0.0, float(NB - 1))
            ti = jnp.minimum(t.astype(jnp.int32), NB - 2)
            f = t - ti.astype(jnp.float32)
            c0 = plsc.load_gather(a_ref, [ti])
            d = plsc.load_gather(b_ref, [ti])
            bufout[pl.ds(j * L, L)] = c0 + f * d

        plsc.parallel_loop(0, CHUNK_R // L, 1, unroll=UNROLL_REMAP)(_remap_body)
        odescs[cur] = pltpu.async_copy(
            bufout, out_hbm.at[pl.ds(off, CHUNK_R)], osems[cur])
        if ci + 2 < NCHUNK_R:
            idescs[cur] = pltpu.async_copy(
                x_hbm.at[pl.ds(base + (ci + 2) * CHUNK_R, CHUNK_R)],
                bufin, isems[cur])

    for d in odescs:
        if d is not None:
            d.wait()


def kernel(x):
    xt = x.reshape(H // 8, 8, W // 128, 128).transpose(0, 2, 1, 3).reshape(-1)
    mm = _minmax(xt)
    hp = _hist(xt, mm)
    y = _remap(xt, hp, mm)
    return (y.reshape(H // 8, W // 128, 8, 128)
             .transpose(0, 2, 1, 3).reshape(1, H, W))
